# baseline (speedup 1.0000x reference)
import functools

import jax
import jax.numpy as jnp
from jax import lax
from jax.experimental import pallas as pl
from jax.experimental.pallas import tpu as pltpu
from jax.experimental.pallas import tpu_sc as plsc

NC = 2
NS = 16
NW = NC * NS

B = 4096
D = 64
CHUNK = 128
NBUF = 4
DG = 3
NTB = 2


def _make_gather(n_rows: int, n_l: int):
  assert n_rows == n_l * B and n_rows % (NW * CHUNK * NBUF) == 0
  per_w = n_rows // NW
  n_chunks = per_w // CHUNK

  mesh = plsc.VectorSubcoreMesh(
      core_axis_name="c", subcore_axis_name="s", num_cores=NC,
      num_subcores=NS)

  @functools.partial(
      pl.kernel,
      out_type=jax.ShapeDtypeStruct((n_l, D, B), jnp.float32),
      mesh=mesh,
      compiler_params=pltpu.CompilerParams(
          use_tc_tiling_on_sc=False, needs_layout_passes=False),
      scratch_types=[
          pltpu.VMEM((per_w,), jnp.int32),
          [pltpu.VMEM((CHUNK, D), jnp.float32) for _ in range(NBUF)],
          [pltpu.VMEM((D, CHUNK), jnp.float32) for _ in range(NTB)],
          [pltpu.SemaphoreType.DMA for _ in range(NBUF)],
          [pltpu.SemaphoreType.DMA for _ in range(NTB)],
      ],
  )
  def gather_kernel(idx_hbm, table_hbm, out_hbm, idx_v, rows, tblk,
                    gsems, ssems):
    wid = lax.axis_index("s") * NC + lax.axis_index("c")
    base = wid * per_w
    pltpu.sync_copy(idx_hbm.at[pl.ds(base, per_w)], idx_v)

    def gather(g, buf):
      return pltpu.make_async_copy(
          table_hbm.at[idx_v.at[pl.ds(g * CHUNK, CHUNK)]], rows[buf],
          gsems[buf])

    def store(g, tb):
      t0 = base + g * CHUNK
      return pltpu.make_async_copy(
          tblk[tb],
          out_hbm.at[t0 // B, :, pl.ds(lax.rem(t0, B), CHUNK)],
          ssems[tb])

    def transpose(buf, tb):
      def body(e2, _):
        for sub in range(2):
          e = e2 * 2 + sub
          col = jnp.full((16,), 0, jnp.int32) + e
          for j in range(CHUNK // 16):
            ridx = lax.iota(jnp.int32, 16) + (j * 16)
            vals = plsc.load_gather(rows[buf], [ridx, col])
            tblk[tb][e, pl.ds(j * 16, 16)] = vals
        return 0

      lax.fori_loop(0, D // 2, body, 0)

    for h in range(DG):
      gather(h, h % NBUF).start()

    def body(grp, _):
      for j in range(NBUF):
        g = grp * NBUF + j
        h = g + DG
        bh = (j + DG) % NBUF

        @pl.when(h < n_chunks)
        def _():
          gather(h, bh).start()

        gather(g, j).wait()
        tb = j % NTB

        @pl.when(g >= NTB)
        def _():
          store(g - NTB, tb).wait()

        transpose(j, tb)
        store(g, tb).start()
      return 0

    lax.fori_loop(0, n_chunks // NBUF, body, 0)

    for j in range(NTB):
      store(n_chunks - NTB + j, (n_chunks - NTB + j) % NTB).wait()

  return gather_kernel


def kernel(x, table):
  b, l = x.shape
  n = b * l
  flat = jnp.transpose(x).reshape(n).astype(jnp.int32)
  out = _make_gather(n, l)(flat, table)
  return jnp.transpose(out, (2, 0, 1))
